# baseline (speedup 1.0000x reference)
import functools
import math

import jax
import jax.numpy as jnp
from jax.experimental import pallas as pl
from jax.experimental.pallas import tpu as pltpu

_NUM_HEADS = 8


def _body(q_ref, oh_ref, ohr_ref, k_ref, v_ref, wq_ref, bq_ref, wk_ref,
          bk_ref, wv_ref, bv_ref, wo_ref, bo_ref, out_ref,
          e_sc, kp_sc, vp_sc, den_sc, *, rs):
    p = pl.program_id(0)
    t = pl.program_id(1)
    f32 = jnp.float32

    @pl.when((p == 0) & (t == 0))
    def _init():
        kp = jnp.dot(k_ref[...], wk_ref[...], preferred_element_type=f32)
        kp_sc[...] = (kp + bk_ref[...]) * rs
        vp = jnp.dot(v_ref[...], wv_ref[...], preferred_element_type=f32)
        vp_sc[...] = vp + bv_ref[...]
        den_sc[...] = jnp.zeros_like(den_sc)

    @pl.when(p == 0)
    def _pass1():
        qp = jnp.dot(q_ref[...], wq_ref[...], preferred_element_type=f32)
        qp = qp + bq_ref[...]
        kg = jnp.dot(oh_ref[...], kp_sc[...], preferred_element_type=f32)
        e = jnp.exp(qp * kg)
        e_sc[t] = e
        den_sc[...] += jnp.dot(ohr_ref[...], e, preferred_element_type=f32)

    @pl.when((p == 1) & (t == 0))
    def _fold():
        den = den_sc[...]
        den_sc[...] = vp_sc[...] / jnp.where(den == 0.0, 1.0, den)

    @pl.when(p == 1)
    def _pass2():
        wg = jnp.dot(oh_ref[...], den_sc[...], preferred_element_type=f32)
        e = e_sc[t]
        out = jnp.dot(e * wg, wo_ref[...], preferred_element_type=f32)
        out_ref[...] = out + bo_ref[...]


def kernel(q, k, v, batch, Wq, bq, Wk, bk, Wv, bv, Wo, bo):
    n, c = q.shape
    nseg = k.shape[0]
    rs = 1.0 / math.sqrt(c // _NUM_HEADS)
    tile = 4096
    nt = n // tile

    seg_ids = jnp.arange(nseg, dtype=batch.dtype)
    oh = (batch[:, None] == seg_ids[None, :]).astype(jnp.float32)
    ohr = (seg_ids[:, None] == batch[None, :]).astype(jnp.float32)

    small = pl.BlockSpec((nseg, c), lambda p, t: (0, 0))
    wspec = pl.BlockSpec((c, c), lambda p, t: (0, 0))
    bspec = pl.BlockSpec((1, c), lambda p, t: (0, 0))

    body = functools.partial(_body, rs=rs)
    out = pl.pallas_call(
        body,
        grid=(2, nt),
        in_specs=[
            pl.BlockSpec((tile, c),
                         lambda p, t: (jnp.where(p == 0, t, nt - 1), 0)),
            pl.BlockSpec((tile, nseg), lambda p, t: (t, 0)),
            pl.BlockSpec((nseg, tile),
                         lambda p, t: (0, jnp.where(p == 0, t, nt - 1))),
            small,
            small,
            wspec, bspec,
            wspec, bspec,
            wspec, bspec,
            wspec, bspec,
        ],
        out_specs=pl.BlockSpec((tile, c),
                               lambda p, t: (jnp.where(p == 0, 0, t), 0)),
        out_shape=jax.ShapeDtypeStruct((n, c), jnp.float32),
        scratch_shapes=[
            pltpu.VMEM((nt, tile, c), jnp.float32),
            pltpu.VMEM((nseg, c), jnp.float32),
            pltpu.VMEM((nseg, c), jnp.float32),
            pltpu.VMEM((nseg, c), jnp.float32),
        ],
    )(q, oh, ohr, k, v,
      Wq.T, bq.reshape(1, c),
      Wk.T, bk.reshape(1, c),
      Wv.T, bv.reshape(1, c),
      Wo.T, bo.reshape(1, c))
    return out


# speedup vs baseline: 1.2043x; 1.2043x over previous
import functools
import math

import jax
import jax.numpy as jnp
from jax.experimental import pallas as pl
from jax.experimental.pallas import tpu as pltpu

_NUM_HEADS = 8


def _body(q_ref, bc_ref, br_ref, k_ref, v_ref, wq_ref, bq_ref, wk_ref,
          bk_ref, wv_ref, bv_ref, wo_ref, bo_ref, out_ref,
          e_sc, kp_sc, vp_sc, den_sc, *, nseg, rs):
    p = pl.program_id(0)
    t = pl.program_id(1)
    f32 = jnp.float32

    @pl.when((p == 0) & (t == 0))
    def _init():
        kp = jnp.dot(k_ref[...], wk_ref[...], preferred_element_type=f32)
        kp_sc[...] = (kp + bk_ref[...]) * rs
        vp = jnp.dot(v_ref[...], wv_ref[...], preferred_element_type=f32)
        vp_sc[...] = vp + bv_ref[...]
        den_sc[...] = jnp.zeros_like(den_sc)

    @pl.when(p == 0)
    def _pass1():
        qp = jnp.dot(q_ref[...], wq_ref[...], preferred_element_type=f32)
        qp = qp + bq_ref[...]
        oh = (bc_ref[...] == jax.lax.broadcasted_iota(
            jnp.int32, (1, nseg), 1)).astype(f32)
        kg = jnp.dot(oh, kp_sc[...], preferred_element_type=f32)
        e = jnp.exp(qp * kg)
        e_sc[t] = e
        oht = (br_ref[...] == jax.lax.broadcasted_iota(
            jnp.int32, (nseg, 1), 0)).astype(f32)
        den_sc[...] += jnp.dot(oht, e, preferred_element_type=f32)

    @pl.when((p == 1) & (t == 0))
    def _fold():
        den = den_sc[...]
        den_sc[...] = vp_sc[...] / jnp.where(den == 0.0, 1.0, den)

    @pl.when(p == 1)
    def _pass2():
        oh = (bc_ref[...] == jax.lax.broadcasted_iota(
            jnp.int32, (1, nseg), 1)).astype(f32)
        wg = jnp.dot(oh, den_sc[...], preferred_element_type=f32)
        e = e_sc[t]
        out = jnp.dot(e * wg, wo_ref[...], preferred_element_type=f32)
        out_ref[...] = out + bo_ref[...]


def kernel(q, k, v, batch, Wq, bq, Wk, bk, Wv, bv, Wo, bo):
    n, c = q.shape
    nseg = k.shape[0]
    rs = 1.0 / math.sqrt(c // _NUM_HEADS)
    tile = 8192
    nt = n // tile

    bc = batch.reshape(n, 1)
    br = batch.reshape(1, n)

    small = pl.BlockSpec((nseg, c), lambda p, t: (0, 0))
    wspec = pl.BlockSpec((c, c), lambda p, t: (0, 0))
    bspec = pl.BlockSpec((1, c), lambda p, t: (0, 0))

    body = functools.partial(_body, nseg=nseg, rs=rs)
    out = pl.pallas_call(
        body,
        grid=(2, nt),
        in_specs=[
            pl.BlockSpec((tile, c),
                         lambda p, t: (jnp.where(p == 0, t, nt - 1), 0)),
            pl.BlockSpec((tile, 1), lambda p, t: (t, 0)),
            pl.BlockSpec((1, tile),
                         lambda p, t: (0, jnp.where(p == 0, t, nt - 1))),
            small,
            small,
            wspec, bspec,
            wspec, bspec,
            wspec, bspec,
            wspec, bspec,
        ],
        out_specs=pl.BlockSpec((tile, c),
                               lambda p, t: (jnp.where(p == 0, 0, t), 0)),
        out_shape=jax.ShapeDtypeStruct((n, c), jnp.float32),
        scratch_shapes=[
            pltpu.VMEM((nt, tile, c), jnp.float32),
            pltpu.VMEM((nseg, c), jnp.float32),
            pltpu.VMEM((nseg, c), jnp.float32),
            pltpu.VMEM((nseg, c), jnp.float32),
        ],
    )(q, bc, br, k, v,
      Wq.T, bq.reshape(1, c),
      Wk.T, bk.reshape(1, c),
      Wv.T, bv.reshape(1, c),
      Wo.T, bo.reshape(1, c))
    return out
